# baseline (speedup 1.0000x reference)
import functools

import jax
import jax.numpy as jnp
from jax import lax
from jax.experimental import pallas as pl
from jax.experimental.pallas import tpu as pltpu
from jax.experimental.pallas import tpu_sc as plsc

N_U = 5000
N_V = 5000
N = N_U + N_V
E = 320000
D = 128

NC = 2
NS = 16
NW = NC * NS
CHUNK = 128
NCHUNK = 80
HNCHUNK = 160
E_PAD = NW * NCHUNK * CHUNK
N_PAD = 10240
ROWS_PER_TILE = N_PAD // NS
HWIN = 8

_mesh = plsc.VectorSubcoreMesh(core_axis_name="c", subcore_axis_name="s")


@functools.partial(
    pl.kernel,
    out_type=jax.ShapeDtypeStruct((NC, N_PAD), jnp.float32),
    mesh=_mesh,
    scratch_types=[
        pltpu.VMEM((HNCHUNK, CHUNK), jnp.int32),
        pltpu.VMEM((CHUNK,), jnp.float32),
        pltpu.VMEM((ROWS_PER_TILE,), jnp.float32),
        pltpu.VMEM_SHARED((N_PAD,), jnp.float32),
        pltpu.SemaphoreType.DMA,
    ],
)
def _hist_kernel(edges_hbm, out_hbm, idx_v, ones_v, zbuf, hist, sem):
    c = lax.axis_index("c")
    s = lax.axis_index("s")

    for k in range(CHUNK // 16):
        ones_v[pl.ds(k * 16, 16)] = jnp.ones((16,), jnp.float32)

    def _zero_body(i, _):
        zbuf[pl.ds(i * 16, 16)] = jnp.zeros((16,), jnp.float32)
        return 0

    lax.fori_loop(0, ROWS_PER_TILE // 16, _zero_body, 0)
    pltpu.sync_copy(zbuf, hist.at[pl.ds(s * ROWS_PER_TILE, ROWS_PER_TILE)])
    plsc.subcore_barrier()

    pltpu.sync_copy(edges_hbm.at[c, s], idx_v)

    pend = []
    for j in range(HNCHUNK):
        if j >= HWIN:
            pend[j - HWIN].wait()
        pend.append(
            pltpu.async_copy(ones_v, hist.at[idx_v.at[j]], sem, add=True))
    for j in range(HNCHUNK - HWIN, HNCHUNK):
        pend[j].wait()
    plsc.subcore_barrier()

    pltpu.sync_copy(hist.at[pl.ds(s * ROWS_PER_TILE, ROWS_PER_TILE)], zbuf)
    pltpu.sync_copy(zbuf, out_hbm.at[c, pl.ds(s * ROWS_PER_TILE, ROWS_PER_TILE)])


@functools.partial(
    pl.kernel,
    out_type=jax.ShapeDtypeStruct((NC, N_PAD, D), jnp.float32),
    mesh=_mesh,
    scratch_types=[
        pltpu.VMEM((NCHUNK, CHUNK), jnp.int32),
        pltpu.VMEM((CHUNK,), jnp.int32),
        pltpu.VMEM((CHUNK,), jnp.int32),
        pltpu.VMEM((CHUNK, D), jnp.float32),
        pltpu.VMEM((CHUNK, D), jnp.float32),
        pltpu.VMEM_SHARED((N_PAD, D), jnp.float32),
        pltpu.SemaphoreType.DMA,
        pltpu.SemaphoreType.DMA,
        pltpu.SemaphoreType.DMA,
        pltpu.SemaphoreType.DMA,
    ],
)
def _gather_scatter_kernel(xs_hbm, src_hbm, dst_hbm, out_hbm,
                           src_v, dst0, dst1, rows0, rows1, acc,
                           semg0, semg1, semd0, semd1):
    c = lax.axis_index("c")
    s = lax.axis_index("s")
    wid = c * NS + s
    bufs = (rows0, rows1)
    dsts = (dst0, dst1)
    semg = (semg0, semg1)
    semd = (semd0, semd1)

    def _zero_body(i, _):
        for k in range(D // 16):
            rows0[i, pl.ds(k * 16, 16)] = jnp.zeros((16,), jnp.float32)
        return 0

    lax.fori_loop(0, CHUNK, _zero_body, 0)
    for k in range(ROWS_PER_TILE // CHUNK):
        pltpu.sync_copy(
            rows0, acc.at[pl.ds(s * ROWS_PER_TILE + k * CHUNK, CHUNK)])
    plsc.subcore_barrier()

    pltpu.sync_copy(src_hbm.at[wid], src_v)

    pend = {}
    for b in range(2):
        pend[b] = (
            pltpu.async_copy(xs_hbm.at[src_v.at[b]], bufs[b], semg[b]),
            pltpu.async_copy(dst_hbm.at[wid, b], dsts[b], semd[b]),
        )

    for jb in range(NCHUNK):
        b = jb % 2
        g, dl = pend[b]
        g.wait()
        dl.wait()
        pltpu.sync_copy(bufs[b], acc.at[dsts[b]], add=True)
        if jb + 2 < NCHUNK:
            pend[b] = (
                pltpu.async_copy(xs_hbm.at[src_v.at[jb + 2]], bufs[b],
                                 semg[b]),
                pltpu.async_copy(dst_hbm.at[wid, jb + 2], dsts[b], semd[b]),
            )
    plsc.subcore_barrier()

    for k in range(ROWS_PER_TILE // CHUNK):
        base = s * ROWS_PER_TILE + k * CHUNK
        pltpu.sync_copy(acc.at[pl.ds(base, CHUNK)], rows0)
        pltpu.sync_copy(rows0, out_hbm.at[c, pl.ds(base, CHUNK)])


def _scale_body(x_ref, d_ref, o_ref):
    norm = lax.rsqrt(jnp.maximum(d_ref[...], 1.0))
    o_ref[...] = x_ref[...] * norm


def _scale_rows(x, d, blk):
    n = x.shape[0]
    return pl.pallas_call(
        _scale_body,
        grid=(n // blk,),
        in_specs=[
            pl.BlockSpec((blk, D), lambda i: (i, 0)),
            pl.BlockSpec((blk, 1), lambda i: (i, 0)),
        ],
        out_specs=pl.BlockSpec((blk, D), lambda i: (i, 0)),
        out_shape=jax.ShapeDtypeStruct((n, D), jnp.float32),
    )(x, d)


def _combine_body(p0_ref, p1_ref, d_ref, o_ref):
    norm = lax.rsqrt(jnp.maximum(d_ref[...], 1.0))
    o_ref[...] = (p0_ref[0] + p1_ref[0]) * norm


def _combine(partials, d, blk):
    return pl.pallas_call(
        _combine_body,
        grid=(N // blk,),
        in_specs=[
            pl.BlockSpec((1, blk, D), lambda i: (0, i, 0)),
            pl.BlockSpec((1, blk, D), lambda i: (1, i, 0)),
            pl.BlockSpec((blk, 1), lambda i: (i, 0)),
        ],
        out_specs=pl.BlockSpec((blk, D), lambda i: (i, 0)),
        out_shape=jax.ShapeDtypeStruct((N, D), jnp.float32),
    )(partials, partials, d)


def kernel(u_f, v_f, edge_index):
    x = jnp.concatenate([u_f, v_f], axis=0)

    npad = E_PAD - E
    padv = N + (jnp.arange(npad, dtype=jnp.int32) % (N_PAD - N))
    e_p = jnp.concatenate(
        [edge_index, jnp.broadcast_to(padv, (2, npad))], axis=1)
    hist_view = e_p.reshape(2, NS, HNCHUNK, CHUNK)
    padv_src = (jnp.arange(npad, dtype=jnp.int32) * 13) % N
    src_p = jnp.concatenate([edge_index[0], padv_src]).reshape(
        NW, NCHUNK, CHUNK)
    dst_p = e_p[1].reshape(NW, NCHUNK, CHUNK)

    hist = _hist_kernel(hist_view)
    outd = hist[0, :N].reshape(N, 1)
    xs = _scale_rows(x, outd, blk=1000)

    partials = _gather_scatter_kernel(xs, src_p, dst_p)

    ind = hist[1, :N].reshape(N, 1)
    return _combine(partials, ind, blk=2000)


# speedup vs baseline: 1.0154x; 1.0154x over previous
import functools

import jax
import jax.numpy as jnp
from jax import lax
from jax.experimental import pallas as pl
from jax.experimental.pallas import tpu as pltpu
from jax.experimental.pallas import tpu_sc as plsc

N_U = 5000
N_V = 5000
N = N_U + N_V
E = 320000
D = 128

NC = 2
NS = 16
NW = NC * NS
CHUNK = 128
NCHUNK = 80
HNCHUNK = 160
E_PAD = NW * NCHUNK * CHUNK
N_PAD = 10240
ROWS_PER_TILE = N_PAD // NS
HWIN = 8

_mesh = plsc.VectorSubcoreMesh(core_axis_name="c", subcore_axis_name="s")


@functools.partial(
    pl.kernel,
    out_type=jax.ShapeDtypeStruct((NC, N_PAD), jnp.float32),
    mesh=_mesh,
    scratch_types=[
        pltpu.VMEM((HNCHUNK, CHUNK), jnp.int32),
        pltpu.VMEM((CHUNK,), jnp.float32),
        pltpu.VMEM((ROWS_PER_TILE,), jnp.float32),
        pltpu.VMEM_SHARED((N_PAD,), jnp.float32),
        pltpu.SemaphoreType.DMA,
    ],
)
def _hist_kernel(edges_hbm, out_hbm, idx_v, ones_v, zbuf, hist, sem):
    c = lax.axis_index("c")
    s = lax.axis_index("s")

    for k in range(CHUNK // 16):
        ones_v[pl.ds(k * 16, 16)] = jnp.ones((16,), jnp.float32)

    def _zero_body(i, _):
        zbuf[pl.ds(i * 16, 16)] = jnp.zeros((16,), jnp.float32)
        return 0

    lax.fori_loop(0, ROWS_PER_TILE // 16, _zero_body, 0)
    pltpu.sync_copy(zbuf, hist.at[pl.ds(s * ROWS_PER_TILE, ROWS_PER_TILE)])
    plsc.subcore_barrier()

    pltpu.sync_copy(edges_hbm.at[c, s], idx_v)

    def _wait_one():
        pltpu.make_async_copy(ones_v, hist.at[idx_v.at[0]], sem).wait()

    for j in range(HWIN):
        pltpu.async_copy(ones_v, hist.at[idx_v.at[j]], sem, add=True)

    def _body(j, _):
        _wait_one()
        pltpu.async_copy(ones_v, hist.at[idx_v.at[j]], sem, add=True)
        return 0

    lax.fori_loop(HWIN, HNCHUNK, _body, 0)
    for _ in range(HWIN):
        _wait_one()
    plsc.subcore_barrier()

    pltpu.sync_copy(hist.at[pl.ds(s * ROWS_PER_TILE, ROWS_PER_TILE)], zbuf)
    pltpu.sync_copy(zbuf, out_hbm.at[c, pl.ds(s * ROWS_PER_TILE, ROWS_PER_TILE)])


@functools.partial(
    pl.kernel,
    out_type=jax.ShapeDtypeStruct((NC, N_PAD, D), jnp.float32),
    mesh=_mesh,
    scratch_types=[
        pltpu.VMEM((NCHUNK, CHUNK), jnp.int32),
        pltpu.VMEM((CHUNK,), jnp.int32),
        pltpu.VMEM((CHUNK,), jnp.int32),
        pltpu.VMEM((CHUNK, D), jnp.float32),
        pltpu.VMEM((CHUNK, D), jnp.float32),
        pltpu.VMEM_SHARED((N_PAD, D), jnp.float32),
        pltpu.SemaphoreType.DMA,
        pltpu.SemaphoreType.DMA,
        pltpu.SemaphoreType.DMA,
        pltpu.SemaphoreType.DMA,
    ],
)
def _gather_scatter_kernel(xs_hbm, src_hbm, dst_hbm, out_hbm,
                           src_v, dst0, dst1, rows0, rows1, acc,
                           semg0, semg1, semd0, semd1):
    c = lax.axis_index("c")
    s = lax.axis_index("s")
    wid = c * NS + s
    bufs = (rows0, rows1)
    dsts = (dst0, dst1)
    semg = (semg0, semg1)
    semd = (semd0, semd1)

    def _zero_body(i, _):
        for k in range(D // 16):
            rows0[i, pl.ds(k * 16, 16)] = jnp.zeros((16,), jnp.float32)
        return 0

    lax.fori_loop(0, CHUNK, _zero_body, 0)
    for k in range(ROWS_PER_TILE // CHUNK):
        pltpu.sync_copy(
            rows0, acc.at[pl.ds(s * ROWS_PER_TILE + k * CHUNK, CHUNK)])
    plsc.subcore_barrier()

    pltpu.sync_copy(src_hbm.at[wid], src_v)

    def _wait_and_scatter(jb, b):
        pltpu.make_async_copy(xs_hbm.at[src_v.at[jb]], bufs[b],
                              semg[b]).wait()
        pltpu.make_async_copy(dst_hbm.at[wid, jb], dsts[b], semd[b]).wait()
        pltpu.sync_copy(bufs[b], acc.at[dsts[b]], add=True)

    def _issue(jb, b):
        pltpu.async_copy(xs_hbm.at[src_v.at[jb]], bufs[b], semg[b])
        pltpu.async_copy(dst_hbm.at[wid, jb], dsts[b], semd[b])

    for b in range(2):
        _issue(b, b)

    def _body(j2, _):
        for b in range(2):
            jb = j2 * 2 + b
            _wait_and_scatter(jb, b)
            _issue(jb + 2, b)
        return 0

    lax.fori_loop(0, NCHUNK // 2 - 1, _body, 0)
    for b in range(2):
        _wait_and_scatter(NCHUNK - 2 + b, b)
    plsc.subcore_barrier()

    for k in range(ROWS_PER_TILE // CHUNK):
        base = s * ROWS_PER_TILE + k * CHUNK
        pltpu.sync_copy(acc.at[pl.ds(base, CHUNK)], rows0)
        pltpu.sync_copy(rows0, out_hbm.at[c, pl.ds(base, CHUNK)])


def _scale_body(x_ref, d_ref, o_ref):
    norm = lax.rsqrt(jnp.maximum(d_ref[...], 1.0))
    o_ref[...] = x_ref[...] * norm


def _scale_rows(x, d, blk):
    n = x.shape[0]
    return pl.pallas_call(
        _scale_body,
        grid=(n // blk,),
        in_specs=[
            pl.BlockSpec((blk, D), lambda i: (i, 0)),
            pl.BlockSpec((blk, 1), lambda i: (i, 0)),
        ],
        out_specs=pl.BlockSpec((blk, D), lambda i: (i, 0)),
        out_shape=jax.ShapeDtypeStruct((n, D), jnp.float32),
    )(x, d)


def _combine_body(p0_ref, p1_ref, d_ref, o_ref):
    norm = lax.rsqrt(jnp.maximum(d_ref[...], 1.0))
    o_ref[...] = (p0_ref[0] + p1_ref[0]) * norm


def _combine(partials, d, blk):
    return pl.pallas_call(
        _combine_body,
        grid=(N // blk,),
        in_specs=[
            pl.BlockSpec((1, blk, D), lambda i: (0, i, 0)),
            pl.BlockSpec((1, blk, D), lambda i: (1, i, 0)),
            pl.BlockSpec((blk, 1), lambda i: (i, 0)),
        ],
        out_specs=pl.BlockSpec((blk, D), lambda i: (i, 0)),
        out_shape=jax.ShapeDtypeStruct((N, D), jnp.float32),
    )(partials, partials, d)


def kernel(u_f, v_f, edge_index):
    x = jnp.concatenate([u_f, v_f], axis=0)

    npad = E_PAD - E
    padv = N + (jnp.arange(npad, dtype=jnp.int32) % (N_PAD - N))
    e_p = jnp.concatenate(
        [edge_index, jnp.broadcast_to(padv, (2, npad))], axis=1)
    hist_view = e_p.reshape(2, NS, HNCHUNK, CHUNK)
    padv_src = (jnp.arange(npad, dtype=jnp.int32) * 13) % N
    src_p = jnp.concatenate([edge_index[0], padv_src]).reshape(
        NW, NCHUNK, CHUNK)
    dst_p = e_p[1].reshape(NW, NCHUNK, CHUNK)

    hist = _hist_kernel(hist_view)
    outd = hist[0, :N].reshape(N, 1)
    xs = _scale_rows(x, outd, blk=1000)

    partials = _gather_scatter_kernel(xs, src_p, dst_p)

    ind = hist[1, :N].reshape(N, 1)
    return _combine(partials, ind, blk=2000)


# speedup vs baseline: 1.1258x; 1.1087x over previous
import functools

import jax
import jax.numpy as jnp
import numpy as np
from jax import lax
from jax.experimental import pallas as pl
from jax.experimental.pallas import tpu as pltpu
from jax.experimental.pallas import tpu_sc as plsc

N_U = 5000
N_V = 5000
N = N_U + N_V
E = 320000
D = 128

NC = 2
NS = 16
NW = NC * NS
CHUNK = 128
NCHUNK = 80
HNCHUNK = 160
E_PAD = NW * NCHUNK * CHUNK
N_PAD = 10240
ROWS_PER_TILE = N_PAD // NS
HWIN = 8

_mesh = plsc.VectorSubcoreMesh(core_axis_name="c", subcore_axis_name="s")


@functools.partial(
    pl.kernel,
    out_type=jax.ShapeDtypeStruct((NC, N_PAD), jnp.float32),
    mesh=_mesh,
    scratch_types=[
        pltpu.VMEM((HNCHUNK, CHUNK), jnp.int32),
        pltpu.VMEM((CHUNK,), jnp.float32),
        pltpu.VMEM((ROWS_PER_TILE,), jnp.float32),
        pltpu.VMEM_SHARED((N_PAD,), jnp.float32),
        pltpu.SemaphoreType.DMA,
    ],
)
def _hist_kernel(edges_hbm, out_hbm, idx_v, ones_v, zbuf, hist, sem):
    c = lax.axis_index("c")
    s = lax.axis_index("s")

    for k in range(CHUNK // 16):
        ones_v[pl.ds(k * 16, 16)] = jnp.ones((16,), jnp.float32)

    def _zero_body(i, _):
        zbuf[pl.ds(i * 16, 16)] = jnp.zeros((16,), jnp.float32)
        return 0

    lax.fori_loop(0, ROWS_PER_TILE // 16, _zero_body, 0)
    pltpu.sync_copy(zbuf, hist.at[pl.ds(s * ROWS_PER_TILE, ROWS_PER_TILE)])
    plsc.subcore_barrier()

    pltpu.sync_copy(edges_hbm.at[c, s], idx_v)

    def _wait_one():
        pltpu.make_async_copy(ones_v, hist.at[idx_v.at[0]], sem).wait()

    for j in range(HWIN):
        pltpu.async_copy(ones_v, hist.at[idx_v.at[j]], sem, add=True)

    def _body(j, _):
        _wait_one()
        pltpu.async_copy(ones_v, hist.at[idx_v.at[j]], sem, add=True)
        return 0

    lax.fori_loop(HWIN, HNCHUNK, _body, 0)
    for _ in range(HWIN):
        _wait_one()
    plsc.subcore_barrier()

    pltpu.sync_copy(hist.at[pl.ds(s * ROWS_PER_TILE, ROWS_PER_TILE)], zbuf)
    pltpu.sync_copy(zbuf, out_hbm.at[c, pl.ds(s * ROWS_PER_TILE, ROWS_PER_TILE)])


@functools.partial(
    pl.kernel,
    out_type=jax.ShapeDtypeStruct((NC, N_PAD, D), jnp.float32),
    mesh=_mesh,
    scratch_types=[
        pltpu.VMEM((NCHUNK, CHUNK), jnp.int32),
        pltpu.VMEM((CHUNK,), jnp.int32),
        pltpu.VMEM((CHUNK,), jnp.int32),
        pltpu.VMEM((CHUNK, D), jnp.float32),
        pltpu.VMEM((CHUNK, D), jnp.float32),
        pltpu.VMEM_SHARED((N_PAD, D), jnp.float32),
        pltpu.SemaphoreType.DMA,
        pltpu.SemaphoreType.DMA,
        pltpu.SemaphoreType.DMA,
        pltpu.SemaphoreType.DMA,
    ],
)
def _gather_scatter_kernel(xs_hbm, src_hbm, dst_hbm, out_hbm,
                           src_v, dst0, dst1, rows0, rows1, acc,
                           semg0, semg1, semd0, semd1):
    c = lax.axis_index("c")
    s = lax.axis_index("s")
    wid = c * NS + s
    bufs = (rows0, rows1)
    dsts = (dst0, dst1)
    semg = (semg0, semg1)
    semd = (semd0, semd1)

    def _zero_body(i, _):
        for k in range(D // 16):
            rows0[i, pl.ds(k * 16, 16)] = jnp.zeros((16,), jnp.float32)
        return 0

    lax.fori_loop(0, CHUNK, _zero_body, 0)
    for k in range(ROWS_PER_TILE // CHUNK):
        pltpu.sync_copy(
            rows0, acc.at[pl.ds(s * ROWS_PER_TILE + k * CHUNK, CHUNK)])
    plsc.subcore_barrier()

    pltpu.sync_copy(src_hbm.at[wid], src_v)

    def _wait_and_scatter(jb, b):
        pltpu.make_async_copy(xs_hbm.at[src_v.at[jb]], bufs[b],
                              semg[b]).wait()
        pltpu.make_async_copy(dst_hbm.at[wid, jb], dsts[b], semd[b]).wait()
        pltpu.sync_copy(bufs[b], acc.at[dsts[b]], add=True)

    def _issue(jb, b):
        pltpu.async_copy(xs_hbm.at[src_v.at[jb]], bufs[b], semg[b])
        pltpu.async_copy(dst_hbm.at[wid, jb], dsts[b], semd[b])

    for b in range(2):
        _issue(b, b)

    def _body(j2, _):
        for b in range(2):
            jb = j2 * 2 + b
            _wait_and_scatter(jb, b)
            _issue(jb + 2, b)
        return 0

    lax.fori_loop(0, NCHUNK // 2 - 1, _body, 0)
    for b in range(2):
        _wait_and_scatter(NCHUNK - 2 + b, b)
    plsc.subcore_barrier()

    for k in range(ROWS_PER_TILE // CHUNK):
        base = s * ROWS_PER_TILE + k * CHUNK
        pltpu.sync_copy(acc.at[pl.ds(base, CHUNK)], rows0)
        pltpu.sync_copy(rows0, out_hbm.at[c, pl.ds(base, CHUNK)])


def _scale_body(x_ref, d_ref, pad_ref, o_ref):
    deg = d_ref[...] - pad_ref[...]
    norm = lax.rsqrt(jnp.maximum(deg, 1.0))
    o_ref[...] = x_ref[...] * norm


def _scale_rows(x, d, padind, blk):
    n = x.shape[0]
    return pl.pallas_call(
        _scale_body,
        grid=(n // blk,),
        in_specs=[
            pl.BlockSpec((blk, D), lambda i: (i, 0)),
            pl.BlockSpec((blk, 1), lambda i: (i, 0)),
            pl.BlockSpec((blk, 1), lambda i: (i, 0)),
        ],
        out_specs=pl.BlockSpec((blk, D), lambda i: (i, 0)),
        out_shape=jax.ShapeDtypeStruct((n, D), jnp.float32),
    )(x, d, padind)


def _combine_body(p0_ref, p1_ref, d_ref, o_ref):
    norm = lax.rsqrt(jnp.maximum(d_ref[...], 1.0))
    o_ref[...] = (p0_ref[0] + p1_ref[0]) * norm


def _combine(partials, d, blk):
    return pl.pallas_call(
        _combine_body,
        grid=(N // blk,),
        in_specs=[
            pl.BlockSpec((1, blk, D), lambda i: (0, i, 0)),
            pl.BlockSpec((1, blk, D), lambda i: (1, i, 0)),
            pl.BlockSpec((blk, 1), lambda i: (i, 0)),
        ],
        out_specs=pl.BlockSpec((blk, D), lambda i: (i, 0)),
        out_shape=jax.ShapeDtypeStruct((N, D), jnp.float32),
    )(partials, partials, d)


def kernel(u_f, v_f, edge_index):
    x = jnp.concatenate([u_f, v_f], axis=0)

    npad = E_PAD - E
    pad_src = (13 * np.arange(npad, dtype=np.int64)) % N
    pad_dst = N + (np.arange(npad, dtype=np.int64) % (N_PAD - N))
    pads = jnp.asarray(np.stack([pad_src, pad_dst]).astype(np.int32))
    e_p = jnp.concatenate([edge_index, pads], axis=1)
    hist_view = e_p.reshape(2, NS, HNCHUNK, CHUNK)
    src_p = e_p[0].reshape(NW, NCHUNK, CHUNK)
    dst_p = e_p[1].reshape(NW, NCHUNK, CHUNK)

    padind_np = np.zeros((N, 1), np.float32)
    padind_np[pad_src] = 1.0
    padind = jnp.asarray(padind_np)

    hist = _hist_kernel(hist_view)
    outd = hist[0, :N].reshape(N, 1)
    xs = _scale_rows(x, outd, padind, blk=1000)

    partials = _gather_scatter_kernel(xs, src_p, dst_p)

    ind = hist[1, :N].reshape(N, 1)
    return _combine(partials, ind, blk=2000)


# speedup vs baseline: 1.1426x; 1.0149x over previous
import functools

import jax
import jax.numpy as jnp
import numpy as np
from jax import lax
from jax.experimental import pallas as pl
from jax.experimental.pallas import tpu as pltpu
from jax.experimental.pallas import tpu_sc as plsc

N_U = 5000
N_V = 5000
N = N_U + N_V
E = 320000
D = 128

NC = 2
NS = 16
NW = NC * NS
CHUNK = 128
NCHUNK = 80
HNCHUNK = 160
E_PAD = NW * NCHUNK * CHUNK
N_PAD = 10240
ROWS_PER_TILE = N_PAD // NS
HWIN = 8

_mesh = plsc.VectorSubcoreMesh(core_axis_name="c", subcore_axis_name="s")


@functools.partial(
    pl.kernel,
    out_type=jax.ShapeDtypeStruct((NC, N_PAD), jnp.float32),
    mesh=_mesh,
    scratch_types=[
        pltpu.VMEM((HNCHUNK, CHUNK), jnp.int32),
        pltpu.VMEM((CHUNK,), jnp.float32),
        pltpu.VMEM((ROWS_PER_TILE,), jnp.float32),
        pltpu.VMEM_SHARED((N_PAD,), jnp.float32),
        pltpu.SemaphoreType.DMA,
    ],
)
def _hist_kernel(edges_hbm, out_hbm, idx_v, ones_v, zbuf, hist, sem):
    c = lax.axis_index("c")
    s = lax.axis_index("s")

    for k in range(CHUNK // 16):
        ones_v[pl.ds(k * 16, 16)] = jnp.ones((16,), jnp.float32)

    def _zero_body(i, _):
        zbuf[pl.ds(i * 16, 16)] = jnp.zeros((16,), jnp.float32)
        return 0

    lax.fori_loop(0, ROWS_PER_TILE // 16, _zero_body, 0)
    pltpu.sync_copy(zbuf, hist.at[pl.ds(s * ROWS_PER_TILE, ROWS_PER_TILE)])
    plsc.subcore_barrier()

    pltpu.sync_copy(edges_hbm.at[c, s], idx_v)

    def _wait_one():
        pltpu.make_async_copy(ones_v, hist.at[idx_v.at[0]], sem).wait()

    for j in range(HWIN):
        pltpu.async_copy(ones_v, hist.at[idx_v.at[j]], sem, add=True)

    def _body(j, _):
        _wait_one()
        pltpu.async_copy(ones_v, hist.at[idx_v.at[j]], sem, add=True)
        return 0

    lax.fori_loop(HWIN, HNCHUNK, _body, 0)
    for _ in range(HWIN):
        _wait_one()
    plsc.subcore_barrier()

    pltpu.sync_copy(hist.at[pl.ds(s * ROWS_PER_TILE, ROWS_PER_TILE)], zbuf)
    pltpu.sync_copy(zbuf, out_hbm.at[c, pl.ds(s * ROWS_PER_TILE, ROWS_PER_TILE)])


@functools.partial(
    pl.kernel,
    out_type=jax.ShapeDtypeStruct((NC, N_PAD, D), jnp.float32),
    mesh=_mesh,
    scratch_types=[
        pltpu.VMEM((NCHUNK, CHUNK), jnp.int32),
        pltpu.VMEM((CHUNK,), jnp.int32),
        pltpu.VMEM((CHUNK,), jnp.int32),
        pltpu.VMEM((CHUNK, D), jnp.float32),
        pltpu.VMEM((CHUNK, D), jnp.float32),
        pltpu.VMEM_SHARED((N_PAD, D), jnp.float32),
        pltpu.SemaphoreType.DMA,
        pltpu.SemaphoreType.DMA,
        pltpu.SemaphoreType.DMA,
        pltpu.SemaphoreType.DMA,
    ],
)
def _gather_scatter_kernel(xs_hbm, src_hbm, dst_hbm, out_hbm,
                           src_v, dst0, dst1, rows0, rows1, acc,
                           semg0, semg1, semd0, semd1):
    c = lax.axis_index("c")
    s = lax.axis_index("s")
    wid = c * NS + s
    bufs = (rows0, rows1)
    dsts = (dst0, dst1)
    semg = (semg0, semg1)
    semd = (semd0, semd1)

    def _zero_body(i, _):
        for k in range(D // 16):
            rows0[i, pl.ds(k * 16, 16)] = jnp.zeros((16,), jnp.float32)
        return 0

    lax.fori_loop(0, CHUNK, _zero_body, 0)
    for k in range(ROWS_PER_TILE // CHUNK):
        pltpu.sync_copy(
            rows0, acc.at[pl.ds(s * ROWS_PER_TILE + k * CHUNK, CHUNK)])
    plsc.subcore_barrier()

    pltpu.sync_copy(src_hbm.at[wid], src_v)

    def _wait_and_scatter(jb, b):
        pltpu.make_async_copy(xs_hbm.at[src_v.at[jb]], bufs[b],
                              semg[b]).wait()
        pltpu.make_async_copy(dst_hbm.at[wid, jb], dsts[b], semd[b]).wait()
        pltpu.sync_copy(bufs[b], acc.at[dsts[b]], add=True)

    def _issue(jb, b):
        pltpu.async_copy(xs_hbm.at[src_v.at[jb]], bufs[b], semg[b])
        pltpu.async_copy(dst_hbm.at[wid, jb], dsts[b], semd[b])

    for b in range(2):
        _issue(b, b)

    def _body(j2, _):
        for b in range(2):
            jb = j2 * 2 + b
            _wait_and_scatter(jb, b)
            _issue(jb + 2, b)
        return 0

    lax.fori_loop(0, NCHUNK // 2 - 1, _body, 0)
    for b in range(2):
        _wait_and_scatter(NCHUNK - 2 + b, b)
    plsc.subcore_barrier()

    for k in range(ROWS_PER_TILE // CHUNK):
        base = s * ROWS_PER_TILE + k * CHUNK
        pltpu.sync_copy(acc.at[pl.ds(base, CHUNK)], rows0)
        pltpu.sync_copy(rows0, out_hbm.at[c, pl.ds(base, CHUNK)])


def _scale_body(x_ref, d_ref, pad_ref, o_ref):
    deg = d_ref[...] - pad_ref[...]
    norm = lax.rsqrt(jnp.maximum(deg, 1.0))
    o_ref[...] = x_ref[...] * norm


def _scale_rows(x, d, padind, blk):
    n = x.shape[0]
    return pl.pallas_call(
        _scale_body,
        grid=(n // blk,),
        in_specs=[
            pl.BlockSpec((blk, D), lambda i: (i, 0)),
            pl.BlockSpec((blk, 1), lambda i: (i, 0)),
            pl.BlockSpec((blk, 1), lambda i: (i, 0)),
        ],
        out_specs=pl.BlockSpec((blk, D), lambda i: (i, 0)),
        out_shape=jax.ShapeDtypeStruct((n, D), jnp.float32),
    )(x, d, padind)


def _combine_body(p0_ref, p1_ref, d_ref, o_ref):
    norm = lax.rsqrt(jnp.maximum(d_ref[...], 1.0))
    o_ref[...] = (p0_ref[0] + p1_ref[0]) * norm


def _combine(partials, d, blk):
    return pl.pallas_call(
        _combine_body,
        grid=(N // blk,),
        in_specs=[
            pl.BlockSpec((1, blk, D), lambda i: (0, i, 0)),
            pl.BlockSpec((1, blk, D), lambda i: (1, i, 0)),
            pl.BlockSpec((blk, 1), lambda i: (i, 0)),
        ],
        out_specs=pl.BlockSpec((blk, D), lambda i: (i, 0)),
        out_shape=jax.ShapeDtypeStruct((N, D), jnp.float32),
    )(partials, partials, d)


def kernel(u_f, v_f, edge_index):
    x = jnp.concatenate([u_f, v_f], axis=0)

    npad = E_PAD - E
    pad_src = (13 * np.arange(npad, dtype=np.int64)) % N
    pad_dst = N + (np.arange(npad, dtype=np.int64) % (N_PAD - N))
    pads = jnp.asarray(np.stack([pad_src, pad_dst]).astype(np.int32))
    e_p = jnp.concatenate([edge_index, pads], axis=1)
    hist_view = e_p.reshape(2, NS, HNCHUNK, CHUNK)
    src_p = e_p[0].reshape(NW, NCHUNK, CHUNK)
    dst_p = e_p[1].reshape(NW, NCHUNK, CHUNK)

    padind_np = np.zeros((N, 1), np.float32)
    padind_np[pad_src] = 1.0
    padind = jnp.asarray(padind_np)

    hist = _hist_kernel(hist_view)
    outd = hist[0, :N].reshape(N, 1)
    xs = _scale_rows(x, outd, padind, blk=2000)

    partials = _gather_scatter_kernel(xs, src_p, dst_p)

    ind = hist[1, :N].reshape(N, 1)
    return _combine(partials, ind, blk=5000)
